# Initial kernel scaffold; baseline (speedup 1.0000x reference)
import jax
import jax.numpy as jnp
from jax.experimental import pallas as pl


def kernel(z_vals, weights, num_samples):
    raise NotImplementedError("write your pallas kernel here")


# speedup vs baseline: 655.7973x; 655.7973x over previous
import jax
import jax.numpy as jnp
from jax import lax
from jax.experimental import pallas as pl
from jax.experimental.pallas import tpu as pltpu
from jax.experimental.pallas import tpu_sc as plsc

N_RAYS_S = 131072
N_COARSE_S = 64
N_FINE_S = 128
L = 16
NUM_CORES = 2
NUM_SUBCORES = 16
NUM_WORKERS = NUM_CORES * NUM_SUBCORES
RAYS_PER_WORKER = N_RAYS_S // NUM_WORKERS
CHUNK = 128
NUM_CHUNKS = RAYS_PER_WORKER // CHUNK
NGC = N_COARSE_S // L
NGF = N_FINE_S // L
CNT_LEN = 144


def _tec_body(z_hbm, w_hbm, out_hbm, zbuf, wbuf, obuf, cdfbuf, cntbuf):
    wid = lax.axis_index("s") * NUM_CORES + lax.axis_index("c")
    base = wid * RAYS_PER_WORKER
    iota_f = lax.broadcasted_iota(jnp.int32, (L,), 0).astype(jnp.float32)
    zero16i = jnp.zeros((L,), jnp.int32)
    one16i = jnp.ones((L,), jnp.int32)

    def do_chunk(ci, chunk_carry):
        row0 = base + ci * CHUNK
        pltpu.sync_copy(z_hbm.at[pl.ds(row0, CHUNK)], zbuf)
        pltpu.sync_copy(w_hbm.at[pl.ds(row0, CHUNK)], wbuf)

        def do_ray(r, ray_carry):
            wv = [wbuf[r, pl.ds(L * g, L)] + 1e-5 for g in range(NGC)]
            s = [jnp.sum(v) for v in wv]
            total = s[0] + s[1] + s[2] + s[3]
            inv = 1.0 / jnp.broadcast_to(total, (L,))
            for g in range(CNT_LEN // L):
                cntbuf[pl.ds(L * g, L)] = zero16i
            carry = jnp.float32(0.0)
            for g in range(NGC):
                cg = (plsc.cumsum(wv[g]) + carry) * inv
                carry = carry + s[g]
                cdfbuf[pl.ds(L * g, L)] = cg
                t = cg * 127.0
                ti = t.astype(jnp.int32)
                pgrid = ti + jnp.where(ti.astype(jnp.float32) < t, one16i, zero16i)
                pgrid = jnp.clip(pgrid, 0, CNT_LEN - 1)
                plsc.addupdate_scatter(cntbuf, [pgrid], one16i)
            r16 = jnp.full((L,), r, jnp.int32)
            icarry = jnp.int32(0)
            for g in range(NGF):
                cc = plsc.cumsum(cntbuf[pl.ds(L * g, L)]) + icarry
                icarry = jnp.max(cc)
                below = cc
                b63 = jnp.minimum(below, 63)
                a63 = jnp.minimum(below + 1, 63)
                z_b = plsc.load_gather(zbuf, [r16, b63])
                z_a = plsc.load_gather(zbuf, [r16, a63])
                cdf_a = plsc.load_gather(cdfbuf, [b63])
                cdf_b = jnp.where(
                    below == 0,
                    jnp.float32(0.0),
                    plsc.load_gather(cdfbuf, [jnp.maximum(below - 1, 0)]),
                )
                u = (iota_f + jnp.float32(L * g)) * jnp.float32(1.0 / 127.0)
                den = cdf_a - cdf_b
                den = jnp.where(den < 1e-5, jnp.float32(1.0), den)
                tt = (u - cdf_b) / den
                obuf[r, pl.ds(L * g, L)] = z_b + tt * (z_a - z_b)
            return ray_carry

        lax.fori_loop(0, CHUNK, do_ray, 0)
        pltpu.sync_copy(obuf, out_hbm.at[pl.ds(row0, CHUNK)])
        return chunk_carry

    lax.fori_loop(0, NUM_CHUNKS, do_chunk, 0)


_sampler = pl.kernel(
    _tec_body,
    out_type=jax.ShapeDtypeStruct((N_RAYS_S, N_FINE_S), jnp.float32),
    mesh=plsc.VectorSubcoreMesh(core_axis_name="c", subcore_axis_name="s"),
    scratch_types=[
        pltpu.VMEM((CHUNK, N_COARSE_S), jnp.float32),
        pltpu.VMEM((CHUNK, N_COARSE_S), jnp.float32),
        pltpu.VMEM((CHUNK, N_FINE_S), jnp.float32),
        pltpu.VMEM((N_COARSE_S,), jnp.float32),
        pltpu.VMEM((CNT_LEN,), jnp.int32),
    ],
    compiler_params=pltpu.CompilerParams(needs_layout_passes=False),
)


def kernel(z_vals, weights, num_samples):
    del num_samples
    return _sampler(z_vals, weights)
